# baseline (speedup 1.0000x reference)
import functools
import numpy as np
import jax
import jax.numpy as jnp
from jax import lax
from jax.experimental import pallas as pl
from jax.experimental.pallas import tpu as pltpu
from jax.experimental.pallas import tpu_sc as plsc

N = 4096
D = 512
L = 4
K = 10
TEMP_INV = 100.0
V_DECAY = 0.95
BN = 256
NB = N // BN
VL = 16
NW = 32
R_TOTAL = L * N
RPW = R_TOTAL // NW
GR = 8
NGROUP = RPW // GR
NCHUNK = N // VL

_DECAY_NORM = float(1.0 / np.sum(V_DECAY ** np.arange(K, dtype=np.float64)))
_LOG_V = float(np.log(V_DECAY))


def _sims_kernel(idx_ref, f_ref, negs_ref, tcol_ref, trow_ref, out_ref):
    l = pl.program_id(0)
    f = f_ref[...]
    fn = f / jnp.maximum(jnp.sqrt(jnp.sum(f * f, axis=1, keepdims=True)), 1e-12)
    g = negs_ref[0]
    gn = g / jnp.maximum(jnp.sqrt(jnp.sum(g * g, axis=1, keepdims=True)), 1e-12)
    scores = jax.lax.dot_general(
        fn, gn, (((1,), (1,)), ((), ())),
        preferred_element_type=jnp.float32,
        precision=jax.lax.Precision.DEFAULT,
    )
    same = tcol_ref[...] == trow_ref[...]
    is_idx = l == idx_ref[0]
    out_ref[...] = jnp.where(jnp.logical_and(is_idx, same),
                             jnp.float32(-jnp.inf), scores)


def _sc_topk_kernel(sims_hbm, out_hbm, rowbuf, outstage):
    cid = lax.axis_index("c")
    sid = lax.axis_index("s")
    wid = sid * 2 + cid
    base_row = wid * RPW

    def merge(t16, c):
        cd = lax.rev(lax.sort(c), (0,))
        m = jnp.maximum(t16, cd)
        return lax.sort(m)

    def do_group(g, _):
        row0 = base_row + g * GR
        pltpu.sync_copy(sims_hbm.at[pl.ds(row0 * N, GR * N)], rowbuf)

        def do_row(r, _):
            roff = r * N

            zeros16 = jnp.zeros((VL,), jnp.int32)

            def do_chunk4(q, carry):
                t16, thr_v = carry
                off = roff + q * (4 * VL)
                c0 = rowbuf[pl.ds(off, VL)]
                c1 = rowbuf[pl.ds(off + VL, VL)]
                c2 = rowbuf[pl.ds(off + 2 * VL, VL)]
                c3 = rowbuf[pl.ds(off + 3 * VL, VL)]
                m = jnp.maximum(jnp.maximum(c0, c1), jnp.maximum(c2, c3))

                def slow():
                    t = t16
                    for c in (c0, c1, c2, c3):
                        t = merge(t, c)
                    thr = lax.gather(
                        t, zeros16[:, None],
                        lax.GatherDimensionNumbers(
                            offset_dims=(), collapsed_slice_dims=(0,),
                            start_index_map=(0,)),
                        slice_sizes=(1,),
                        mode=lax.GatherScatterMode.PROMISE_IN_BOUNDS)
                    return t, thr

                return lax.cond(jnp.any(m > thr_v), slow, lambda: carry)

            t16, _ = lax.fori_loop(
                0, NCHUNK // 4, do_chunk4,
                (jnp.full((VL,), -jnp.inf, jnp.float32),
                 jnp.full((VL,), -jnp.inf, jnp.float32)))
            desc = lax.rev(t16, (0,))
            outstage[pl.ds((g * GR + r) * VL, VL)] = desc
            return 0

        lax.fori_loop(0, GR, do_row, 0)
        return 0

    lax.fori_loop(0, NGROUP, do_group, 0)
    pltpu.sync_copy(outstage, out_hbm.at[pl.ds(base_row * VL, RPW * VL)])


def _entropy_kernel(tops_ref, out_ref):
    x = tops_ref[...]
    m = jnp.max(x, axis=0)
    z = (x - m[None]) * TEMP_INV
    e = jnp.exp(z)
    s1 = jnp.sum(e, axis=0)
    s2 = jnp.sum(e * z, axis=0)
    ent = s2 / s1 - jnp.log(s1)
    lane = jax.lax.broadcasted_iota(jnp.int32, (1, VL), 1)
    decay = jnp.where(lane < K,
                      jnp.exp(lane.astype(jnp.float32) * _LOG_V) * _DECAY_NORM,
                      0.0)
    out_ref[...] = (jnp.sum(ent * decay, keepdims=True).reshape(1, 1)
                    * (1.0 / N) + jnp.log(jnp.float32(L)))


@jax.jit
def _run(feature, target, negative_features, idx):
    idx_s = jnp.asarray(idx, jnp.int32).reshape(1)
    tcol = target.astype(jnp.int32).reshape(N, 1)
    trow = target.astype(jnp.int32).reshape(1, N)

    sims = pl.pallas_call(
        _sims_kernel,
        grid=(L, NB),
        in_specs=[
            pl.BlockSpec(memory_space=pltpu.SMEM),
            pl.BlockSpec((BN, D), lambda l, nb: (nb, 0)),
            pl.BlockSpec((1, N, D), lambda l, nb: (l, 0, 0)),
            pl.BlockSpec((BN, 1), lambda l, nb: (nb, 0)),
            pl.BlockSpec((1, N), lambda l, nb: (0, 0)),
        ],
        out_specs=pl.BlockSpec((BN, N), lambda l, nb: (l * NB + nb, 0)),
        out_shape=jax.ShapeDtypeStruct((R_TOTAL, N), jnp.float32),
    )(idx_s, feature, negative_features, tcol, trow)

    mesh = plsc.VectorSubcoreMesh(core_axis_name="c", subcore_axis_name="s",
                                  num_cores=2, num_subcores=16)
    tops_flat = pl.kernel(
        _sc_topk_kernel,
        out_type=jax.ShapeDtypeStruct((R_TOTAL * VL,), jnp.float32),
        mesh=mesh,
        compiler_params=pltpu.CompilerParams(needs_layout_passes=False),
        scratch_types=[
            pltpu.VMEM((GR * N,), jnp.float32),
            pltpu.VMEM((RPW * VL,), jnp.float32),
        ],
    )(sims.reshape(R_TOTAL * N))

    tops = tops_flat.reshape(L, N, VL)
    out = pl.pallas_call(
        _entropy_kernel,
        grid=(1,),
        in_specs=[pl.BlockSpec((L, N, VL), lambda i: (0, 0, 0))],
        out_specs=pl.BlockSpec((1, 1), lambda i: (0, 0)),
        out_shape=jax.ShapeDtypeStruct((1, 1), jnp.float32),
    )(tops)
    return out[0, 0]


def kernel(feature, target, negative_features, idx):
    return _run(feature, target, negative_features, idx)


# speedup vs baseline: 1.4847x; 1.4847x over previous
import functools
import numpy as np
import jax
import jax.numpy as jnp
from jax import lax
from jax.experimental import pallas as pl
from jax.experimental.pallas import tpu as pltpu
from jax.experimental.pallas import tpu_sc as plsc

N = 4096
D = 512
L = 4
K = 10
TEMP_INV = 100.0
V_DECAY = 0.95
BN = 256
NB = N // BN
VL = 16
NW = 32
R_TOTAL = L * N
RPW = R_TOTAL // NW
GR = 8
NGROUP = RPW // GR
NCHUNK = N // VL

_DECAY_NORM = float(1.0 / np.sum(V_DECAY ** np.arange(K, dtype=np.float64)))
_LOG_V = float(np.log(V_DECAY))


def _sims_kernel(idx_ref, f_ref, negs_ref, tcol_ref, trow_ref, out_ref):
    l = pl.program_id(0)
    f = f_ref[...]
    fn = f / jnp.maximum(jnp.sqrt(jnp.sum(f * f, axis=1, keepdims=True)), 1e-12)
    g = negs_ref[0]
    gn = g / jnp.maximum(jnp.sqrt(jnp.sum(g * g, axis=1, keepdims=True)), 1e-12)
    scores = jax.lax.dot_general(
        fn, gn, (((1,), (1,)), ((), ())),
        preferred_element_type=jnp.float32,
        precision=jax.lax.Precision.DEFAULT,
    )
    same = tcol_ref[...] == trow_ref[...]
    is_idx = l == idx_ref[0]
    out_ref[...] = jnp.where(jnp.logical_and(is_idx, same),
                             jnp.float32(-jnp.inf), scores)


def _sc_topk_kernel(sims_hbm, out_hbm, rowbuf, outstage):
    cid = lax.axis_index("c")
    sid = lax.axis_index("s")
    wid = sid * 2 + cid
    base_row = wid * RPW

    def merge(t16, c):
        cd = lax.rev(lax.sort(c), (0,))
        m = jnp.maximum(t16, cd)
        return lax.sort(m)

    def do_group(g, _):
        row0 = base_row + g * GR
        pltpu.sync_copy(sims_hbm.at[pl.ds(row0 * N, GR * N)], rowbuf)

        def do_row(r, _):
            roff = r * N

            def do_chunk8(q, t16):
                off = roff + q * (8 * VL)
                for s in range(8):
                    t16 = merge(t16, rowbuf[pl.ds(off + s * VL, VL)])
                return t16

            t16 = lax.fori_loop(0, NCHUNK // 8, do_chunk8,
                                jnp.full((VL,), -jnp.inf, jnp.float32))
            desc = lax.rev(t16, (0,))
            outstage[pl.ds((g * GR + r) * VL, VL)] = desc
            return 0

        lax.fori_loop(0, GR, do_row, 0)
        return 0

    lax.fori_loop(0, NGROUP, do_group, 0)
    pltpu.sync_copy(outstage, out_hbm.at[pl.ds(base_row * VL, RPW * VL)])


def _entropy_kernel(tops_ref, out_ref):
    x = tops_ref[...]
    m = jnp.max(x, axis=0)
    z = (x - m[None]) * TEMP_INV
    e = jnp.exp(z)
    s1 = jnp.sum(e, axis=0)
    s2 = jnp.sum(e * z, axis=0)
    ent = s2 / s1 - jnp.log(s1)
    lane = jax.lax.broadcasted_iota(jnp.int32, (1, VL), 1)
    decay = jnp.where(lane < K,
                      jnp.exp(lane.astype(jnp.float32) * _LOG_V) * _DECAY_NORM,
                      0.0)
    out_ref[...] = (jnp.sum(ent * decay, keepdims=True).reshape(1, 1)
                    * (1.0 / N) + jnp.log(jnp.float32(L)))


@jax.jit
def _run(feature, target, negative_features, idx):
    idx_s = jnp.asarray(idx, jnp.int32).reshape(1)
    tcol = target.astype(jnp.int32).reshape(N, 1)
    trow = target.astype(jnp.int32).reshape(1, N)

    sims = pl.pallas_call(
        _sims_kernel,
        grid=(L, NB),
        in_specs=[
            pl.BlockSpec(memory_space=pltpu.SMEM),
            pl.BlockSpec((BN, D), lambda l, nb: (nb, 0)),
            pl.BlockSpec((1, N, D), lambda l, nb: (l, 0, 0)),
            pl.BlockSpec((BN, 1), lambda l, nb: (nb, 0)),
            pl.BlockSpec((1, N), lambda l, nb: (0, 0)),
        ],
        out_specs=pl.BlockSpec((BN, N), lambda l, nb: (l * NB + nb, 0)),
        out_shape=jax.ShapeDtypeStruct((R_TOTAL, N), jnp.float32),
    )(idx_s, feature, negative_features, tcol, trow)

    mesh = plsc.VectorSubcoreMesh(core_axis_name="c", subcore_axis_name="s",
                                  num_cores=2, num_subcores=16)
    tops_flat = pl.kernel(
        _sc_topk_kernel,
        out_type=jax.ShapeDtypeStruct((R_TOTAL * VL,), jnp.float32),
        mesh=mesh,
        compiler_params=pltpu.CompilerParams(needs_layout_passes=False),
        scratch_types=[
            pltpu.VMEM((GR * N,), jnp.float32),
            pltpu.VMEM((RPW * VL,), jnp.float32),
        ],
    )(sims.reshape(R_TOTAL * N))

    tops = tops_flat.reshape(L, N, VL)
    out = pl.pallas_call(
        _entropy_kernel,
        grid=(1,),
        in_specs=[pl.BlockSpec((L, N, VL), lambda i: (0, 0, 0))],
        out_specs=pl.BlockSpec((1, 1), lambda i: (0, 0)),
        out_shape=jax.ShapeDtypeStruct((1, 1), jnp.float32),
    )(tops)
    return out[0, 0]


def kernel(feature, target, negative_features, idx):
    return _run(feature, target, negative_features, idx)


# speedup vs baseline: 1.6655x; 1.1218x over previous
import functools
import numpy as np
import jax
import jax.numpy as jnp
from jax import lax
from jax.experimental import pallas as pl
from jax.experimental.pallas import tpu as pltpu
from jax.experimental.pallas import tpu_sc as plsc

N = 4096
D = 512
L = 4
K = 10
TEMP_INV = 100.0
V_DECAY = 0.95
BN = 256
NB = N // BN
VL = 16
NW = 32
R_TOTAL = L * N
RPW = R_TOTAL // NW
GR = 8
NGROUP = RPW // GR
NCHUNK = N // VL

_DECAY_NORM = float(1.0 / np.sum(V_DECAY ** np.arange(K, dtype=np.float64)))
_LOG_V = float(np.log(V_DECAY))


def _sims_kernel(idx_ref, f_ref, negs_ref, tcol_ref, trow_ref, out_ref):
    l = pl.program_id(0)
    f = f_ref[...]
    fn = f / jnp.maximum(jnp.sqrt(jnp.sum(f * f, axis=1, keepdims=True)), 1e-12)
    g = negs_ref[0]
    gn = g / jnp.maximum(jnp.sqrt(jnp.sum(g * g, axis=1, keepdims=True)), 1e-12)
    scores = jax.lax.dot_general(
        fn, gn, (((1,), (1,)), ((), ())),
        preferred_element_type=jnp.float32,
        precision=jax.lax.Precision.DEFAULT,
    )
    same = tcol_ref[...] == trow_ref[...]
    is_idx = l == idx_ref[0]
    out_ref[...] = jnp.where(jnp.logical_and(is_idx, same),
                             jnp.float32(-jnp.inf), scores)


def _sc_topk_kernel(sims_hbm, out_hbm, rowbuf, outstage):
    cid = lax.axis_index("c")
    sid = lax.axis_index("s")
    wid = sid * 2 + cid
    base_row = wid * RPW

    def merge(t16, c):
        cd = lax.rev(lax.sort(c), (0,))
        m = jnp.maximum(t16, cd)
        return lax.sort(m)

    def do_group(g, _):
        row0 = base_row + g * GR
        pltpu.sync_copy(sims_hbm.at[pl.ds(row0, GR)], rowbuf)

        def do_row(r, _):
            def do_chunk8(q, t16):
                off = q * (8 * VL)
                for s in range(8):
                    t16 = merge(t16, rowbuf[r, pl.ds(off + s * VL, VL)])
                return t16

            t16 = lax.fori_loop(0, NCHUNK // 8, do_chunk8,
                                jnp.full((VL,), -jnp.inf, jnp.float32))
            desc = lax.rev(t16, (0,))
            outstage[pl.ds((g * GR + r) * VL, VL)] = desc
            return 0

        lax.fori_loop(0, GR, do_row, 0)
        return 0

    lax.fori_loop(0, NGROUP, do_group, 0)
    pltpu.sync_copy(outstage, out_hbm.at[pl.ds(base_row * VL, RPW * VL)])


def _entropy_kernel(tops_ref, out_ref):
    x = tops_ref[...]
    m = jnp.max(x, axis=0)
    z = (x - m[None]) * TEMP_INV
    e = jnp.exp(z)
    s1 = jnp.sum(e, axis=0)
    s2 = jnp.sum(e * z, axis=0)
    ent = s2 / s1 - jnp.log(s1)
    lane = jax.lax.broadcasted_iota(jnp.int32, (1, VL), 1)
    decay = jnp.where(lane < K,
                      jnp.exp(lane.astype(jnp.float32) * _LOG_V) * _DECAY_NORM,
                      0.0)
    out_ref[...] = (jnp.sum(ent * decay, keepdims=True).reshape(1, 1)
                    * (1.0 / N) + jnp.log(jnp.float32(L)))


@jax.jit
def _run(feature, target, negative_features, idx):
    idx_s = jnp.asarray(idx, jnp.int32).reshape(1)
    tcol = target.astype(jnp.int32).reshape(N, 1)
    trow = target.astype(jnp.int32).reshape(1, N)

    sims = pl.pallas_call(
        _sims_kernel,
        grid=(L, NB),
        in_specs=[
            pl.BlockSpec(memory_space=pltpu.SMEM),
            pl.BlockSpec((BN, D), lambda l, nb: (nb, 0)),
            pl.BlockSpec((1, N, D), lambda l, nb: (l, 0, 0)),
            pl.BlockSpec((BN, 1), lambda l, nb: (nb, 0)),
            pl.BlockSpec((1, N), lambda l, nb: (0, 0)),
        ],
        out_specs=pl.BlockSpec((BN, N), lambda l, nb: (l * NB + nb, 0)),
        out_shape=jax.ShapeDtypeStruct((R_TOTAL, N), jnp.float32),
    )(idx_s, feature, negative_features, tcol, trow)

    mesh = plsc.VectorSubcoreMesh(core_axis_name="c", subcore_axis_name="s",
                                  num_cores=2, num_subcores=16)
    tops_flat = pl.kernel(
        _sc_topk_kernel,
        out_type=jax.ShapeDtypeStruct((R_TOTAL * VL,), jnp.float32),
        mesh=mesh,
        compiler_params=pltpu.CompilerParams(needs_layout_passes=False),
        scratch_types=[
            pltpu.VMEM((GR, N), jnp.float32),
            pltpu.VMEM((RPW * VL,), jnp.float32),
        ],
    )(sims)

    tops = tops_flat.reshape(L, N, VL)
    out = pl.pallas_call(
        _entropy_kernel,
        grid=(1,),
        in_specs=[pl.BlockSpec((L, N, VL), lambda i: (0, 0, 0))],
        out_specs=pl.BlockSpec((1, 1), lambda i: (0, 0)),
        out_shape=jax.ShapeDtypeStruct((1, 1), jnp.float32),
    )(tops)
    return out[0, 0]


def kernel(feature, target, negative_features, idx):
    return _run(feature, target, negative_features, idx)


# speedup vs baseline: 1.6836x; 1.0108x over previous
import functools
import numpy as np
import jax
import jax.numpy as jnp
from jax import lax
from jax.experimental import pallas as pl
from jax.experimental.pallas import tpu as pltpu
from jax.experimental.pallas import tpu_sc as plsc

N = 4096
D = 512
L = 4
K = 10
TEMP_INV = 100.0
V_DECAY = 0.95
BN = 256
NB = N // BN
VL = 16
NW = 32
R_TOTAL = L * N
RPW = R_TOTAL // NW
GR = 16
NGROUP = RPW // GR
NCHUNK = N // VL

_DECAY_NORM = float(1.0 / np.sum(V_DECAY ** np.arange(K, dtype=np.float64)))
_LOG_V = float(np.log(V_DECAY))


def _sims_kernel(idx_ref, f_ref, negs_ref, tcol_ref, trow_ref, out_ref):
    l = pl.program_id(0)
    f = f_ref[...]
    fn = f / jnp.maximum(jnp.sqrt(jnp.sum(f * f, axis=1, keepdims=True)), 1e-12)
    g = negs_ref[0]
    gn = g / jnp.maximum(jnp.sqrt(jnp.sum(g * g, axis=1, keepdims=True)), 1e-12)
    scores = jax.lax.dot_general(
        fn, gn, (((1,), (1,)), ((), ())),
        preferred_element_type=jnp.float32,
        precision=jax.lax.Precision.DEFAULT,
    )
    same = tcol_ref[...] == trow_ref[...]
    is_idx = l == idx_ref[0]
    out_ref[...] = jnp.where(jnp.logical_and(is_idx, same),
                             jnp.float32(-jnp.inf), scores)


def _sc_topk_kernel(sims_hbm, out_hbm, rowbuf, outstage):
    cid = lax.axis_index("c")
    sid = lax.axis_index("s")
    wid = sid * 2 + cid
    base_row = wid * RPW

    def merge(t16, c):
        cd = lax.rev(lax.sort(c), (0,))
        m = jnp.maximum(t16, cd)
        return lax.sort(m)

    def do_group(g, _):
        row0 = base_row + g * GR
        pltpu.sync_copy(sims_hbm.at[pl.ds(row0, GR)], rowbuf)

        def do_row(r, _):
            def do_chunk8(q, t16):
                off = q * (8 * VL)
                for s in range(8):
                    t16 = merge(t16, rowbuf[r, pl.ds(off + s * VL, VL)])
                return t16

            t16 = lax.fori_loop(0, NCHUNK // 8, do_chunk8,
                                jnp.full((VL,), -jnp.inf, jnp.float32))
            desc = lax.rev(t16, (0,))
            outstage[pl.ds((g * GR + r) * VL, VL)] = desc
            return 0

        lax.fori_loop(0, GR, do_row, 0)
        return 0

    lax.fori_loop(0, NGROUP, do_group, 0)
    pltpu.sync_copy(outstage, out_hbm.at[pl.ds(base_row * VL, RPW * VL)])


def _entropy_kernel(tops_ref, out_ref):
    x = tops_ref[...]
    m = jnp.max(x, axis=0)
    z = (x - m[None]) * TEMP_INV
    e = jnp.exp(z)
    s1 = jnp.sum(e, axis=0)
    s2 = jnp.sum(e * z, axis=0)
    ent = s2 / s1 - jnp.log(s1)
    lane = jax.lax.broadcasted_iota(jnp.int32, (1, VL), 1)
    decay = jnp.where(lane < K,
                      jnp.exp(lane.astype(jnp.float32) * _LOG_V) * _DECAY_NORM,
                      0.0)
    out_ref[...] = (jnp.sum(ent * decay, keepdims=True).reshape(1, 1)
                    * (1.0 / N) + jnp.log(jnp.float32(L)))


@jax.jit
def _run(feature, target, negative_features, idx):
    idx_s = jnp.asarray(idx, jnp.int32).reshape(1)
    tcol = target.astype(jnp.int32).reshape(N, 1)
    trow = target.astype(jnp.int32).reshape(1, N)

    sims = pl.pallas_call(
        _sims_kernel,
        grid=(L, NB),
        in_specs=[
            pl.BlockSpec(memory_space=pltpu.SMEM),
            pl.BlockSpec((BN, D), lambda l, nb: (nb, 0)),
            pl.BlockSpec((1, N, D), lambda l, nb: (l, 0, 0)),
            pl.BlockSpec((BN, 1), lambda l, nb: (nb, 0)),
            pl.BlockSpec((1, N), lambda l, nb: (0, 0)),
        ],
        out_specs=pl.BlockSpec((BN, N), lambda l, nb: (l * NB + nb, 0)),
        out_shape=jax.ShapeDtypeStruct((R_TOTAL, N), jnp.float32),
    )(idx_s, feature, negative_features, tcol, trow)

    mesh = plsc.VectorSubcoreMesh(core_axis_name="c", subcore_axis_name="s",
                                  num_cores=2, num_subcores=16)
    tops_flat = pl.kernel(
        _sc_topk_kernel,
        out_type=jax.ShapeDtypeStruct((R_TOTAL * VL,), jnp.float32),
        mesh=mesh,
        compiler_params=pltpu.CompilerParams(needs_layout_passes=False),
        scratch_types=[
            pltpu.VMEM((GR, N), jnp.float32),
            pltpu.VMEM((RPW * VL,), jnp.float32),
        ],
    )(sims)

    tops = tops_flat.reshape(L, N, VL)
    out = pl.pallas_call(
        _entropy_kernel,
        grid=(1,),
        in_specs=[pl.BlockSpec((L, N, VL), lambda i: (0, 0, 0))],
        out_specs=pl.BlockSpec((1, 1), lambda i: (0, 0)),
        out_shape=jax.ShapeDtypeStruct((1, 1), jnp.float32),
    )(tops)
    return out[0, 0]


def kernel(feature, target, negative_features, idx):
    return _run(feature, target, negative_features, idx)


# speedup vs baseline: 1.7997x; 1.0689x over previous
import functools
import numpy as np
import jax
import jax.numpy as jnp
from jax import lax
from jax.experimental import pallas as pl
from jax.experimental.pallas import tpu as pltpu
from jax.experimental.pallas import tpu_sc as plsc

N = 4096
D = 512
L = 4
K = 10
TEMP_INV = 100.0
V_DECAY = 0.95
BN = 256
NB = N // BN
VL = 16
NW = 32
R_TOTAL = L * N
RPW = R_TOTAL // NW
GR = 8
NGROUP = RPW // GR
NCHUNK = N // VL

_DECAY_NORM = float(1.0 / np.sum(V_DECAY ** np.arange(K, dtype=np.float64)))
_LOG_V = float(np.log(V_DECAY))


def _sims_kernel(idx_ref, f_ref, negs_ref, tcol_ref, trow_ref, out_ref):
    l = pl.program_id(0)
    f = f_ref[...]
    fn = f / jnp.maximum(jnp.sqrt(jnp.sum(f * f, axis=1, keepdims=True)), 1e-12)
    g = negs_ref[0]
    gn = g / jnp.maximum(jnp.sqrt(jnp.sum(g * g, axis=1, keepdims=True)), 1e-12)
    scores = jax.lax.dot_general(
        fn, gn, (((1,), (1,)), ((), ())),
        preferred_element_type=jnp.float32,
        precision=jax.lax.Precision.DEFAULT,
    )
    same = tcol_ref[...] == trow_ref[...]
    is_idx = l == idx_ref[0]
    out_ref[...] = jnp.where(jnp.logical_and(is_idx, same),
                             jnp.float32(-jnp.inf), scores)


def _sc_topk_kernel(sims_hbm, out_hbm, rowbuf, outstage, sem0, sem1):
    cid = lax.axis_index("c")
    sid = lax.axis_index("s")
    wid = sid * 2 + cid
    base_row = wid * RPW

    def merge(t16, c):
        cd = lax.rev(lax.sort(c), (0,))
        m = jnp.maximum(t16, cd)
        return lax.sort(m)

    def start_fetch(g, b, sem):
        row0 = base_row + jnp.minimum(g, NGROUP - 1) * GR
        pltpu.async_copy(sims_hbm.at[pl.ds(row0, GR)], rowbuf.at[b], sem)

    def wait_fetch(b, sem):
        pltpu.make_async_copy(sims_hbm.at[pl.ds(0, GR)], rowbuf.at[b],
                              sem).wait()

    def process(g, b):
        def do_row(r, _):
            def do_chunk8(q, t16):
                off = q * (8 * VL)
                for s in range(8):
                    t16 = merge(t16, rowbuf[b, r, pl.ds(off + s * VL, VL)])
                return t16

            t16 = lax.fori_loop(0, NCHUNK // 8, do_chunk8,
                                jnp.full((VL,), -jnp.inf, jnp.float32))
            desc = lax.rev(t16, (0,))
            outstage[pl.ds((g * GR + r) * VL, VL)] = desc
            return 0

        lax.fori_loop(0, GR, do_row, 0)

    start_fetch(0, 0, sem0)

    def do_pair(p, _):
        g0 = 2 * p
        start_fetch(g0 + 1, 1, sem1)
        wait_fetch(0, sem0)
        process(g0, 0)
        start_fetch(g0 + 2, 0, sem0)
        wait_fetch(1, sem1)
        process(g0 + 1, 1)
        return 0

    lax.fori_loop(0, NGROUP // 2, do_pair, 0)
    wait_fetch(0, sem0)
    pltpu.sync_copy(outstage, out_hbm.at[pl.ds(base_row * VL, RPW * VL)])


def _entropy_kernel(tops_ref, out_ref):
    x = tops_ref[...]
    m = jnp.max(x, axis=0)
    z = (x - m[None]) * TEMP_INV
    e = jnp.exp(z)
    s1 = jnp.sum(e, axis=0)
    s2 = jnp.sum(e * z, axis=0)
    ent = s2 / s1 - jnp.log(s1)
    lane = jax.lax.broadcasted_iota(jnp.int32, (1, VL), 1)
    decay = jnp.where(lane < K,
                      jnp.exp(lane.astype(jnp.float32) * _LOG_V) * _DECAY_NORM,
                      0.0)
    out_ref[...] = (jnp.sum(ent * decay, keepdims=True).reshape(1, 1)
                    * (1.0 / N) + jnp.log(jnp.float32(L)))


@jax.jit
def _run(feature, target, negative_features, idx):
    idx_s = jnp.asarray(idx, jnp.int32).reshape(1)
    tcol = target.astype(jnp.int32).reshape(N, 1)
    trow = target.astype(jnp.int32).reshape(1, N)

    sims = pl.pallas_call(
        _sims_kernel,
        grid=(L, NB),
        in_specs=[
            pl.BlockSpec(memory_space=pltpu.SMEM),
            pl.BlockSpec((BN, D), lambda l, nb: (nb, 0)),
            pl.BlockSpec((1, N, D), lambda l, nb: (l, 0, 0)),
            pl.BlockSpec((BN, 1), lambda l, nb: (nb, 0)),
            pl.BlockSpec((1, N), lambda l, nb: (0, 0)),
        ],
        out_specs=pl.BlockSpec((BN, N), lambda l, nb: (l * NB + nb, 0)),
        out_shape=jax.ShapeDtypeStruct((R_TOTAL, N), jnp.float32),
    )(idx_s, feature, negative_features, tcol, trow)

    mesh = plsc.VectorSubcoreMesh(core_axis_name="c", subcore_axis_name="s",
                                  num_cores=2, num_subcores=16)
    tops_flat = pl.kernel(
        _sc_topk_kernel,
        out_type=jax.ShapeDtypeStruct((R_TOTAL * VL,), jnp.float32),
        mesh=mesh,
        compiler_params=pltpu.CompilerParams(needs_layout_passes=False),
        scratch_types=[
            pltpu.VMEM((2, GR, N), jnp.float32),
            pltpu.VMEM((RPW * VL,), jnp.float32),
            pltpu.SemaphoreType.DMA,
            pltpu.SemaphoreType.DMA,
        ],
    )(sims)

    tops = tops_flat.reshape(L, N, VL)
    out = pl.pallas_call(
        _entropy_kernel,
        grid=(1,),
        in_specs=[pl.BlockSpec((L, N, VL), lambda i: (0, 0, 0))],
        out_specs=pl.BlockSpec((1, 1), lambda i: (0, 0)),
        out_shape=jax.ShapeDtypeStruct((1, 1), jnp.float32),
    )(tops)
    return out[0, 0]


def kernel(feature, target, negative_features, idx):
    return _run(feature, target, negative_features, idx)


# speedup vs baseline: 1.9281x; 1.0714x over previous
import functools
import numpy as np
import jax
import jax.numpy as jnp
from jax import lax
from jax.experimental import pallas as pl
from jax.experimental.pallas import tpu as pltpu
from jax.experimental.pallas import tpu_sc as plsc

N = 4096
D = 512
L = 4
K = 10
TEMP_INV = 100.0
V_DECAY = 0.95
BN = 256
NB = N // BN
VL = 16
NW = 32
GR = 8
NCHUNK = N // VL

_DECAY_NORM = float(1.0 / np.sum(V_DECAY ** np.arange(K, dtype=np.float64)))
_LOG_V = float(np.log(V_DECAY))


def _make_sims_kernel(l_const):
    def _sims_kernel(idx_ref, f_ref, negs_ref, tcol_ref, trow_ref, out_ref):
        f = f_ref[...]
        fn = f / jnp.maximum(jnp.sqrt(jnp.sum(f * f, axis=1, keepdims=True)),
                             1e-12)
        g = negs_ref[...]
        gn = g / jnp.maximum(jnp.sqrt(jnp.sum(g * g, axis=1, keepdims=True)),
                             1e-12)
        scores = jax.lax.dot_general(
            fn, gn, (((1,), (1,)), ((), ())),
            preferred_element_type=jnp.float32,
            precision=jax.lax.Precision.DEFAULT,
        )
        same = tcol_ref[...] == trow_ref[...]
        is_idx = l_const == idx_ref[0]
        out_ref[...] = jnp.where(jnp.logical_and(is_idx, same),
                                 jnp.float32(-jnp.inf), scores)
    return _sims_kernel


def _make_sc_topk(rows_total):
    rpw = rows_total // NW
    ngroup = rpw // GR

    def _sc_topk_kernel(sims_hbm, out_hbm, rowbuf, outstage, sem0, sem1):
        cid = lax.axis_index("c")
        sid = lax.axis_index("s")
        wid = sid * 2 + cid
        base_row = wid * rpw

        def merge(t16, c):
            cd = lax.rev(lax.sort(c), (0,))
            m = jnp.maximum(t16, cd)
            return lax.sort(m)

        def start_fetch(g, b, sem):
            row0 = base_row + jnp.minimum(g, ngroup - 1) * GR
            pltpu.async_copy(sims_hbm.at[pl.ds(row0, GR)], rowbuf.at[b], sem)

        def wait_fetch(b, sem):
            pltpu.make_async_copy(sims_hbm.at[pl.ds(0, GR)], rowbuf.at[b],
                                  sem).wait()

        def process(g, b):
            def do_row(r, _):
                def do_chunk8(q, t16):
                    off = q * (8 * VL)
                    for s in range(8):
                        t16 = merge(t16,
                                    rowbuf[b, r, pl.ds(off + s * VL, VL)])
                    return t16

                t16 = lax.fori_loop(0, NCHUNK // 8, do_chunk8,
                                    jnp.full((VL,), -jnp.inf, jnp.float32))
                desc = lax.rev(t16, (0,))
                outstage[pl.ds((g * GR + r) * VL, VL)] = desc
                return 0

            lax.fori_loop(0, GR, do_row, 0)

        start_fetch(0, 0, sem0)

        def do_pair(p, _):
            g0 = 2 * p
            start_fetch(g0 + 1, 1, sem1)
            wait_fetch(0, sem0)
            process(g0, 0)
            start_fetch(g0 + 2, 0, sem0)
            wait_fetch(1, sem1)
            process(g0 + 1, 1)
            return 0

        lax.fori_loop(0, ngroup // 2, do_pair, 0)
        wait_fetch(0, sem0)
        pltpu.sync_copy(outstage, out_hbm.at[pl.ds(base_row * VL, rpw * VL)])

    return _sc_topk_kernel


def _entropy_kernel(tops_ref, out_ref):
    x = tops_ref[...]
    m = jnp.max(x, axis=0)
    z = (x - m[None]) * TEMP_INV
    e = jnp.exp(z)
    s1 = jnp.sum(e, axis=0)
    s2 = jnp.sum(e * z, axis=0)
    ent = s2 / s1 - jnp.log(s1)
    lane = jax.lax.broadcasted_iota(jnp.int32, (1, VL), 1)
    decay = jnp.where(lane < K,
                      jnp.exp(lane.astype(jnp.float32) * _LOG_V) * _DECAY_NORM,
                      0.0)
    out_ref[...] = (jnp.sum(ent * decay, keepdims=True).reshape(1, 1)
                    * (1.0 / N) + jnp.log(jnp.float32(L)))


@jax.jit
def _run(feature, target, negative_features, idx):
    idx_s = jnp.asarray(idx, jnp.int32).reshape(1)
    tcol = target.astype(jnp.int32).reshape(N, 1)
    trow = target.astype(jnp.int32).reshape(1, N)

    mesh = plsc.VectorSubcoreMesh(core_axis_name="c", subcore_axis_name="s",
                                  num_cores=2, num_subcores=16)
    rpw = N // NW
    sc_topk = pl.kernel(
        _make_sc_topk(N),
        out_type=jax.ShapeDtypeStruct((N * VL,), jnp.float32),
        mesh=mesh,
        compiler_params=pltpu.CompilerParams(needs_layout_passes=False),
        scratch_types=[
            pltpu.VMEM((2, GR, N), jnp.float32),
            pltpu.VMEM((rpw * VL,), jnp.float32),
            pltpu.SemaphoreType.DMA,
            pltpu.SemaphoreType.DMA,
        ],
    )

    tops_l = []
    for l in range(L):
        sims_l = pl.pallas_call(
            _make_sims_kernel(l),
            grid=(NB,),
            in_specs=[
                pl.BlockSpec(memory_space=pltpu.SMEM),
                pl.BlockSpec((BN, D), lambda nb: (nb, 0)),
                pl.BlockSpec((N, D), lambda nb: (0, 0)),
                pl.BlockSpec((BN, 1), lambda nb: (nb, 0)),
                pl.BlockSpec((1, N), lambda nb: (0, 0)),
            ],
            out_specs=pl.BlockSpec((BN, N), lambda nb: (nb, 0)),
            out_shape=jax.ShapeDtypeStruct((N, N), jnp.float32),
        )(idx_s, feature, negative_features[l], tcol, trow)
        tops_l.append(sc_topk(sims_l).reshape(N, VL))

    tops = jnp.stack(tops_l)
    out = pl.pallas_call(
        _entropy_kernel,
        grid=(1,),
        in_specs=[pl.BlockSpec((L, N, VL), lambda i: (0, 0, 0))],
        out_specs=pl.BlockSpec((1, 1), lambda i: (0, 0)),
        out_shape=jax.ShapeDtypeStruct((1, 1), jnp.float32),
    )(tops)
    return out[0, 0]


def kernel(feature, target, negative_features, idx):
    return _run(feature, target, negative_features, idx)
